# baseline (speedup 1.0000x reference)
import functools

import jax
import jax.numpy as jnp
from jax import lax
from jax.experimental import pallas as pl
from jax.experimental.pallas import tpu as pltpu
from jax.experimental.pallas import tpu_sc as plsc

VOCAB = 100000
EMB = 128
NCHAR = 128
CDIM = 16
FSIZE = 64
FWIDTH = 5
B = 64
DL = 512
QL = 32
WL = 16

NW_TOTAL = B * DL + B * QL


_NC = 2
_NS = 16
_NWK = _NC * _NS
_PER_W = NW_TOTAL // _NWK
_NCHUNK = 17
_CHUNK = _PER_W // _NCHUNK


def _sc_gather_body(tbl_hbm, idx_hbm, out_hbm, idx_v, rows_v, sem):
    wid = lax.axis_index("s") * _NC + lax.axis_index("c")
    pltpu.sync_copy(idx_hbm.at[wid], idx_v)

    def step(c, _):
        pltpu.async_copy(tbl_hbm.at[idx_v.at[c]], rows_v, sem).wait()
        base = wid * _PER_W + c * _CHUNK
        pltpu.sync_copy(rows_v, out_hbm.at[pl.ds(base, _CHUNK)])
        return ()

    lax.fori_loop(0, _NCHUNK, step, (), unroll=False)


@jax.jit
def _sc_gather(table, idx):
    kern = pl.kernel(
        _sc_gather_body,
        out_type=jax.ShapeDtypeStruct((NW_TOTAL, EMB), jnp.float32),
        mesh=plsc.VectorSubcoreMesh(core_axis_name="c", subcore_axis_name="s"),
        scratch_types=[
            pltpu.VMEM((_NCHUNK, _CHUNK), jnp.int32),
            pltpu.VMEM((_CHUNK, EMB), jnp.float32),
            pltpu.SemaphoreType.DMA,
        ],
        compiler_params=pltpu.CompilerParams(use_tc_tiling_on_sc=True),
    )
    return kern(table, idx.reshape(_NWK, _NCHUNK, _CHUNK))


_BW = 1024
_NB = _BW * WL
_GRID = NW_TOTAL // _BW
_KDIM = FWIDTH * NCHAR


def _prep_body(tbl_ref, w80_ref, out_ref):
    tbl = tbl_ref[...]
    out_ref[...] = jnp.concatenate(
        [jnp.dot(tbl, w80_ref[pl.ds(CDIM * d, CDIM), :],
                 preferred_element_type=jnp.float32)
         for d in range(FWIDTH)], axis=0)


@jax.jit
def _tc_prep(char_table, w80):
    return pl.pallas_call(
        _prep_body,
        out_shape=jax.ShapeDtypeStruct((_KDIM, FSIZE), jnp.float32),
    )(char_table, w80)


_NP = WL - FWIDTH + 1


def _make_conv_body(bpb, seq):
    def _conv_body(ids_ref, p_ref, b_ref, w_ref, out_ref):
        iota = lax.broadcasted_iota(jnp.int32, (1, 1, NCHAR), 2)
        ids = ids_ref[...].reshape(_BW, WL)
        oh = (ids[:, :, None] == iota).astype(jnp.bfloat16)
        oh = oh.reshape(_BW, WL * NCHAR)
        m = jnp.concatenate(
            [oh[:, NCHAR * p: NCHAR * p + _KDIM] for p in range(_NP)],
            axis=0)
        y = jnp.dot(m, p_ref[...], preferred_element_type=jnp.float32)
        y = jnp.max(y.reshape(_NP, _BW, FSIZE), axis=0)
        y = jnp.maximum(y + b_ref[...], 0.0)
        full = jnp.concatenate([w_ref[...], y], axis=1)
        out_ref[...] = full.reshape(bpb, seq, EMB + FSIZE)
    return _conv_body


def _tc_charconv(c3, p640, b_row, w_emb, seq, w_off):
    nbatch = c3.shape[0]
    bpb = _BW // seq
    return pl.pallas_call(
        _make_conv_body(bpb, seq),
        out_shape=jax.ShapeDtypeStruct((nbatch, seq, EMB + FSIZE),
                                       jnp.float32),
        grid=(nbatch // bpb,),
        in_specs=[
            pl.BlockSpec((bpb, seq, WL), lambda i: (i, 0, 0)),
            pl.BlockSpec((_KDIM, FSIZE), lambda i: (0, 0)),
            pl.BlockSpec((1, FSIZE), lambda i: (0, 0)),
            pl.BlockSpec((_BW, EMB), lambda i: (i + w_off, 0)),
        ],
        out_specs=pl.BlockSpec((bpb, seq, EMB + FSIZE), lambda i: (i, 0, 0)),
    )(c3, p640.astype(jnp.bfloat16), b_row, w_emb)


def kernel(doc_w, doc_c, qry_w, qry_c, k_layer, K, W, char_table, conv_w, conv_b):
    widx = jnp.concatenate(
        [doc_w.reshape(-1), qry_w.reshape(-1)]).astype(jnp.int32)

    w80 = jnp.transpose(conv_w[:, :, 0, :], (2, 1, 0)).reshape(
        FWIDTH * CDIM, FSIZE)
    b_row = conv_b.reshape(1, FSIZE)

    p640 = _tc_prep(char_table, w80)
    w_emb = _sc_gather(W, widx)
    doc_emb = _tc_charconv(doc_c.astype(jnp.int32), p640, b_row, w_emb,
                           DL, 0)
    qry_emb = _tc_charconv(qry_c.astype(jnp.int32), p640, b_row, w_emb,
                           QL, B * DL // _BW)
    return doc_emb, qry_emb


# speedup vs baseline: 1.0653x; 1.0653x over previous
import functools

import jax
import jax.numpy as jnp
from jax import lax
from jax.experimental import pallas as pl
from jax.experimental.pallas import tpu as pltpu
from jax.experimental.pallas import tpu_sc as plsc

VOCAB = 100000
EMB = 128
NCHAR = 128
CDIM = 16
FSIZE = 64
FWIDTH = 5
B = 64
DL = 512
QL = 32
WL = 16

NW_TOTAL = B * DL + B * QL


_NC = 2
_NS = 16
_NWK = _NC * _NS
_PER_W = NW_TOTAL // _NWK
_NCHUNK = 17
_CHUNK = _PER_W // _NCHUNK


def _sc_gather_body(tbl_hbm, idx_hbm, out_hbm, idx_v, rows_v, sem):
    wid = lax.axis_index("s") * _NC + lax.axis_index("c")
    pltpu.sync_copy(idx_hbm.at[wid], idx_v)

    def step(c, _):
        pltpu.async_copy(tbl_hbm.at[idx_v.at[c]], rows_v, sem).wait()
        base = wid * _PER_W + c * _CHUNK
        pltpu.sync_copy(rows_v, out_hbm.at[pl.ds(base, _CHUNK)])
        return ()

    lax.fori_loop(0, _NCHUNK, step, (), unroll=False)


@jax.jit
def _sc_gather(table, idx):
    kern = pl.kernel(
        _sc_gather_body,
        out_type=jax.ShapeDtypeStruct((NW_TOTAL, EMB), jnp.float32),
        mesh=plsc.VectorSubcoreMesh(core_axis_name="c", subcore_axis_name="s"),
        scratch_types=[
            pltpu.VMEM((_NCHUNK, _CHUNK), jnp.int32),
            pltpu.VMEM((_CHUNK, EMB), jnp.float32),
            pltpu.SemaphoreType.DMA,
        ],
        compiler_params=pltpu.CompilerParams(use_tc_tiling_on_sc=True),
    )
    return kern(table, idx.reshape(_NWK, _NCHUNK, _CHUNK))


_BW = 1024
_NB = _BW * WL
_GRID = NW_TOTAL // _BW
_KDIM = FWIDTH * NCHAR


def _prep_body(tbl_ref, w80_ref, out_ref):
    tbl = tbl_ref[...]
    out_ref[...] = jnp.concatenate(
        [jnp.dot(tbl, w80_ref[pl.ds(CDIM * d, CDIM), :],
                 preferred_element_type=jnp.float32)
         for d in range(FWIDTH)], axis=0)


@jax.jit
def _tc_prep(char_table, w80):
    return pl.pallas_call(
        _prep_body,
        out_shape=jax.ShapeDtypeStruct((_KDIM, FSIZE), jnp.float32),
    )(char_table, w80)


_NP = WL - FWIDTH + 1


def _conv_body(ids_ref, p_ref, b_ref, w_ref, out_ref):
    iota = lax.broadcasted_iota(jnp.int32, (1, 1, NCHAR), 2)
    oh = (ids_ref[...][:, :, None] == iota).astype(jnp.bfloat16)
    oh = oh.reshape(_BW, WL * NCHAR)
    m = jnp.concatenate(
        [oh[:, NCHAR * p: NCHAR * p + _KDIM] for p in range(_NP)],
        axis=0)
    y = jnp.dot(m, p_ref[...], preferred_element_type=jnp.float32)
    y = jnp.max(y.reshape(_NP, _BW, FSIZE), axis=0)
    y = jnp.maximum(y + b_ref[...], 0.0)
    out_ref[...] = jnp.concatenate([w_ref[...], y], axis=1)


def _tc_charconv(cidx, p640, b_row, w_emb, n_words, ids_off, w_off):
    return pl.pallas_call(
        _conv_body,
        out_shape=jax.ShapeDtypeStruct((n_words, EMB + FSIZE), jnp.float32),
        grid=(n_words // _BW,),
        in_specs=[
            pl.BlockSpec((_BW, WL), lambda i: (i + ids_off, 0)),
            pl.BlockSpec((_KDIM, FSIZE), lambda i: (0, 0)),
            pl.BlockSpec((1, FSIZE), lambda i: (0, 0)),
            pl.BlockSpec((_BW, EMB), lambda i: (i + w_off, 0)),
        ],
        out_specs=pl.BlockSpec((_BW, EMB + FSIZE), lambda i: (i, 0)),
    )(cidx, p640.astype(jnp.bfloat16), b_row, w_emb)


def kernel(doc_w, doc_c, qry_w, qry_c, k_layer, K, W, char_table, conv_w, conv_b):
    widx = jnp.concatenate(
        [doc_w.reshape(-1), qry_w.reshape(-1)]).astype(jnp.int32)
    doc_cidx = doc_c.reshape(-1, WL).astype(jnp.int32)
    qry_cidx = qry_c.reshape(-1, WL).astype(jnp.int32)

    w80 = jnp.transpose(conv_w[:, :, 0, :], (2, 1, 0)).reshape(
        FWIDTH * CDIM, FSIZE)
    b_row = conv_b.reshape(1, FSIZE)

    p640 = _tc_prep(char_table, w80)
    w_emb = _sc_gather(W, widx)
    doc_full = _tc_charconv(doc_cidx, p640, b_row, w_emb, B * DL, 0, 0)
    qry_full = _tc_charconv(qry_cidx, p640, b_row, w_emb, B * QL, 0,
                            B * DL // _BW)

    doc_emb = doc_full.reshape(B, DL, EMB + FSIZE)
    qry_emb = qry_full.reshape(B, QL, EMB + FSIZE)
    return doc_emb, qry_emb


# speedup vs baseline: 1.0835x; 1.0171x over previous
import functools

import jax
import jax.numpy as jnp
from jax import lax
from jax.experimental import pallas as pl
from jax.experimental.pallas import tpu as pltpu
from jax.experimental.pallas import tpu_sc as plsc

VOCAB = 100000
EMB = 128
NCHAR = 128
CDIM = 16
FSIZE = 64
FWIDTH = 5
B = 64
DL = 512
QL = 32
WL = 16

NW_TOTAL = B * DL + B * QL


_NC = 2
_NS = 16
_NWK = _NC * _NS
_PER_W = NW_TOTAL // _NWK
_NCHUNK = 17
_CHUNK = _PER_W // _NCHUNK


def _sc_gather_body(tbl_hbm, idx_hbm, out_hbm, idx_v, rows_v, sem):
    wid = lax.axis_index("s") * _NC + lax.axis_index("c")
    pltpu.sync_copy(idx_hbm.at[wid], idx_v)
    pltpu.async_copy(tbl_hbm.at[idx_v.at[0]], rows_v.at[0], sem)

    def step(c, _):
        pltpu.make_async_copy(
            tbl_hbm.at[idx_v.at[c]], rows_v.at[c % 2], sem).wait()

        @pl.when(c < _NCHUNK - 1)
        def _start_next():
            pltpu.async_copy(
                tbl_hbm.at[idx_v.at[c + 1]], rows_v.at[(c + 1) % 2], sem)

        base = wid * _PER_W + c * _CHUNK
        pltpu.sync_copy(rows_v.at[c % 2], out_hbm.at[pl.ds(base, _CHUNK)])
        return ()

    lax.fori_loop(0, _NCHUNK, step, (), unroll=False)


@jax.jit
def _sc_gather(table, idx):
    kern = pl.kernel(
        _sc_gather_body,
        out_type=jax.ShapeDtypeStruct((NW_TOTAL, EMB), jnp.float32),
        mesh=plsc.VectorSubcoreMesh(core_axis_name="c", subcore_axis_name="s"),
        scratch_types=[
            pltpu.VMEM((_NCHUNK, _CHUNK), jnp.int32),
            pltpu.VMEM((2, _CHUNK, EMB), jnp.float32),
            pltpu.SemaphoreType.DMA,
        ],
        compiler_params=pltpu.CompilerParams(use_tc_tiling_on_sc=True),
    )
    return kern(table, idx.reshape(_NWK, _NCHUNK, _CHUNK))


_BW = 1024
_NB = _BW * WL
_GRID = NW_TOTAL // _BW
_KDIM = FWIDTH * NCHAR


def _prep_body(tbl_ref, w80_ref, out_ref):
    tbl = tbl_ref[...]
    out_ref[...] = jnp.concatenate(
        [jnp.dot(tbl, w80_ref[pl.ds(CDIM * d, CDIM), :],
                 preferred_element_type=jnp.float32)
         for d in range(FWIDTH)], axis=0)


@jax.jit
def _tc_prep(char_table, w80):
    return pl.pallas_call(
        _prep_body,
        out_shape=jax.ShapeDtypeStruct((_KDIM, FSIZE), jnp.float32),
    )(char_table, w80)


_NP = WL - FWIDTH + 1


def _conv_body(ids_ref, p_ref, b_ref, w_ref, out_ref):
    iota = lax.broadcasted_iota(jnp.int32, (1, 1, NCHAR), 2)
    oh = (ids_ref[...][:, :, None] == iota).astype(jnp.bfloat16)
    oh = oh.reshape(_BW, WL * NCHAR)
    m = jnp.concatenate(
        [oh[:, NCHAR * p: NCHAR * p + _KDIM] for p in range(_NP)],
        axis=0)
    y = jnp.dot(m, p_ref[...], preferred_element_type=jnp.float32)
    y = jnp.max(y.reshape(_NP, _BW, FSIZE), axis=0)
    y = jnp.maximum(y + b_ref[...], 0.0)
    out_ref[...] = jnp.concatenate([w_ref[...], y], axis=1)


def _tc_charconv(cidx, p640, b_row, w_emb, n_words, ids_off, w_off):
    return pl.pallas_call(
        _conv_body,
        out_shape=jax.ShapeDtypeStruct((n_words, EMB + FSIZE), jnp.float32),
        grid=(n_words // _BW,),
        in_specs=[
            pl.BlockSpec((_BW, WL), lambda i: (i + ids_off, 0)),
            pl.BlockSpec((_KDIM, FSIZE), lambda i: (0, 0)),
            pl.BlockSpec((1, FSIZE), lambda i: (0, 0)),
            pl.BlockSpec((_BW, EMB), lambda i: (i + w_off, 0)),
        ],
        out_specs=pl.BlockSpec((_BW, EMB + FSIZE), lambda i: (i, 0)),
    )(cidx, p640.astype(jnp.bfloat16), b_row, w_emb)


def kernel(doc_w, doc_c, qry_w, qry_c, k_layer, K, W, char_table, conv_w, conv_b):
    widx = jnp.concatenate(
        [doc_w.reshape(-1), qry_w.reshape(-1)]).astype(jnp.int32)
    doc_cidx = doc_c.reshape(-1, WL).astype(jnp.int32)
    qry_cidx = qry_c.reshape(-1, WL).astype(jnp.int32)

    w80 = jnp.transpose(conv_w[:, :, 0, :], (2, 1, 0)).reshape(
        FWIDTH * CDIM, FSIZE)
    b_row = conv_b.reshape(1, FSIZE)

    p640 = _tc_prep(char_table, w80)
    w_emb = _sc_gather(W, widx)
    doc_full = _tc_charconv(doc_cidx, p640, b_row, w_emb, B * DL, 0, 0)
    qry_full = _tc_charconv(qry_cidx, p640, b_row, w_emb, B * QL, 0,
                            B * DL // _BW)

    doc_emb = doc_full.reshape(B, DL, EMB + FSIZE)
    qry_emb = qry_full.reshape(B, QL, EMB + FSIZE)
    return doc_emb, qry_emb


# speedup vs baseline: 1.3676x; 1.2622x over previous
import functools

import jax
import jax.numpy as jnp
from jax import lax
from jax.experimental import pallas as pl
from jax.experimental.pallas import tpu as pltpu
from jax.experimental.pallas import tpu_sc as plsc

VOCAB = 100000
EMB = 128
NCHAR = 128
CDIM = 16
FSIZE = 64
FWIDTH = 5
B = 64
DL = 512
QL = 32
WL = 16

NW_TOTAL = B * DL + B * QL


_NC = 2
_NS = 16
_NWK = _NC * _NS
_PER_W = NW_TOTAL // _NWK
_NCHUNK = 17
_CHUNK = _PER_W // _NCHUNK


def _sc_gather_body(tbl_hbm, idx_hbm, out_hbm, idx_v, rows_v, sem):
    wid = lax.axis_index("s") * _NC + lax.axis_index("c")
    pltpu.sync_copy(idx_hbm.at[wid], idx_v)
    pltpu.async_copy(tbl_hbm.at[idx_v.at[0]], rows_v.at[0], sem)

    def step(c, _):
        pltpu.make_async_copy(
            tbl_hbm.at[idx_v.at[c]], rows_v.at[c % 2], sem).wait()

        @pl.when(c < _NCHUNK - 1)
        def _start_next():
            pltpu.async_copy(
                tbl_hbm.at[idx_v.at[c + 1]], rows_v.at[(c + 1) % 2], sem)

        base = wid * _PER_W + c * _CHUNK
        pltpu.sync_copy(rows_v.at[c % 2], out_hbm.at[pl.ds(base, _CHUNK)])
        return ()

    lax.fori_loop(0, _NCHUNK, step, (), unroll=False)


@jax.jit
def _sc_gather(table, idx):
    kern = pl.kernel(
        _sc_gather_body,
        out_type=jax.ShapeDtypeStruct((NW_TOTAL, EMB), jnp.float32),
        mesh=plsc.VectorSubcoreMesh(core_axis_name="c", subcore_axis_name="s"),
        scratch_types=[
            pltpu.VMEM((_NCHUNK, _CHUNK), jnp.int32),
            pltpu.VMEM((2, _CHUNK, EMB), jnp.float32),
            pltpu.SemaphoreType.DMA,
        ],
        compiler_params=pltpu.CompilerParams(use_tc_tiling_on_sc=True),
    )
    return kern(table, idx.reshape(_NWK, _NCHUNK, _CHUNK))


_BW = 1024
_NB = _BW * WL
_GRID = NW_TOTAL // _BW
_KDIM = FWIDTH * NCHAR


_NP = WL - FWIDTH + 1
_G = 2
_NSLAB = _NP // _G
_KSLAB = (FWIDTH - 1 + _G) * NCHAR


def _prep_body(tbl_ref, w80_ref, out_ref):
    tbl = tbl_ref[...]
    p640 = jnp.concatenate(
        [jnp.dot(tbl, w80_ref[pl.ds(CDIM * d, CDIM), :],
                 preferred_element_type=jnp.float32)
         for d in range(FWIDTH)], axis=0)
    cols = []
    for q in range(_G):
        parts = []
        if q:
            parts.append(jnp.zeros((NCHAR * q, FSIZE), jnp.float32))
        parts.append(p640)
        tail = _KSLAB - NCHAR * q - _KDIM
        if tail:
            parts.append(jnp.zeros((tail, FSIZE), jnp.float32))
        cols.append(jnp.concatenate(parts, axis=0))
    out_ref[...] = jnp.concatenate(cols, axis=1)


@jax.jit
def _tc_prep(char_table, w80):
    return pl.pallas_call(
        _prep_body,
        out_shape=jax.ShapeDtypeStruct((_KSLAB, _G * FSIZE), jnp.float32),
    )(char_table, w80)


def _conv_body(ids_ref, p_ref, b_ref, w_ref, out_ref):
    iota = lax.broadcasted_iota(jnp.int32, (1, 1, NCHAR), 2)
    oh = (ids_ref[...][:, :, None] == iota).astype(jnp.bfloat16)
    oh = oh.reshape(_BW, WL * NCHAR)
    m = jnp.concatenate(
        [oh[:, NCHAR * _G * j: NCHAR * _G * j + _KSLAB]
         for j in range(_NSLAB)], axis=0)
    y = jnp.dot(m, p_ref[...], preferred_element_type=jnp.float32)
    y = jnp.max(y.reshape(_NSLAB, _BW, _G * FSIZE), axis=0)
    yq = y[:, :FSIZE]
    for q in range(1, _G):
        yq = jnp.maximum(yq, y[:, FSIZE * q: FSIZE * (q + 1)])
    y = yq
    y = jnp.maximum(y + b_ref[...], 0.0)
    out_ref[...] = jnp.concatenate([w_ref[...], y], axis=1)


def _tc_charconv(cidx, p640, b_row, w_emb, n_words, ids_off, w_off):
    return pl.pallas_call(
        _conv_body,
        out_shape=jax.ShapeDtypeStruct((n_words, EMB + FSIZE), jnp.float32),
        grid=(n_words // _BW,),
        in_specs=[
            pl.BlockSpec((_BW, WL), lambda i: (i + ids_off, 0)),
            pl.BlockSpec((_KSLAB, _G * FSIZE), lambda i: (0, 0)),
            pl.BlockSpec((1, FSIZE), lambda i: (0, 0)),
            pl.BlockSpec((_BW, EMB), lambda i: (i + w_off, 0)),
        ],
        out_specs=pl.BlockSpec((_BW, EMB + FSIZE), lambda i: (i, 0)),
    )(cidx, p640.astype(jnp.bfloat16), b_row, w_emb)


def kernel(doc_w, doc_c, qry_w, qry_c, k_layer, K, W, char_table, conv_w, conv_b):
    widx = jnp.concatenate(
        [doc_w.reshape(-1), qry_w.reshape(-1)]).astype(jnp.int32)
    doc_cidx = doc_c.reshape(-1, WL).astype(jnp.int32)
    qry_cidx = qry_c.reshape(-1, WL).astype(jnp.int32)

    w80 = jnp.transpose(conv_w[:, :, 0, :], (2, 1, 0)).reshape(
        FWIDTH * CDIM, FSIZE)
    b_row = conv_b.reshape(1, FSIZE)

    p640 = _tc_prep(char_table, w80)
    w_emb = _sc_gather(W, widx)
    doc_full = _tc_charconv(doc_cidx, p640, b_row, w_emb, B * DL, 0, 0)
    qry_full = _tc_charconv(qry_cidx, p640, b_row, w_emb, B * QL, 0,
                            B * DL // _BW)

    doc_emb = doc_full.reshape(B, DL, EMB + FSIZE)
    qry_emb = qry_full.reshape(B, QL, EMB + FSIZE)
    return doc_emb, qry_emb


# speedup vs baseline: 1.4027x; 1.0257x over previous
import functools

import jax
import jax.numpy as jnp
from jax import lax
from jax.experimental import pallas as pl
from jax.experimental.pallas import tpu as pltpu
from jax.experimental.pallas import tpu_sc as plsc

VOCAB = 100000
EMB = 128
NCHAR = 128
CDIM = 16
FSIZE = 64
FWIDTH = 5
B = 64
DL = 512
QL = 32
WL = 16

NW_TOTAL = B * DL + B * QL


_NC = 2
_NS = 16
_NWK = _NC * _NS
_PER_W = NW_TOTAL // _NWK
_NCHUNK = 17
_CHUNK = _PER_W // _NCHUNK


def _sc_gather_body(tbl_hbm, idx_hbm, out_hbm, idx_v, rows_v, sem):
    wid = lax.axis_index("s") * _NC + lax.axis_index("c")
    pltpu.sync_copy(idx_hbm.at[wid], idx_v)
    pltpu.async_copy(tbl_hbm.at[idx_v.at[0]], rows_v.at[0], sem)

    def step(c, _):
        pltpu.make_async_copy(
            tbl_hbm.at[idx_v.at[c]], rows_v.at[c % 2], sem).wait()

        @pl.when(c < _NCHUNK - 1)
        def _start_next():
            pltpu.async_copy(
                tbl_hbm.at[idx_v.at[c + 1]], rows_v.at[(c + 1) % 2], sem)

        base = wid * _PER_W + c * _CHUNK
        pltpu.sync_copy(rows_v.at[c % 2], out_hbm.at[pl.ds(base, _CHUNK)])
        return ()

    lax.fori_loop(0, _NCHUNK, step, (), unroll=False)


@jax.jit
def _sc_gather(table, idx):
    kern = pl.kernel(
        _sc_gather_body,
        out_type=jax.ShapeDtypeStruct((NW_TOTAL, EMB), jnp.float32),
        mesh=plsc.VectorSubcoreMesh(core_axis_name="c", subcore_axis_name="s"),
        scratch_types=[
            pltpu.VMEM((_NCHUNK, _CHUNK), jnp.int32),
            pltpu.VMEM((2, _CHUNK, EMB), jnp.float32),
            pltpu.SemaphoreType.DMA,
        ],
        compiler_params=pltpu.CompilerParams(use_tc_tiling_on_sc=True),
    )
    return kern(table, idx.reshape(_NWK, _NCHUNK, _CHUNK))


_BW = 2048
_NB = _BW * WL
_GRID = NW_TOTAL // _BW
_KDIM = FWIDTH * NCHAR


_NP = WL - FWIDTH + 1
_G = 2
_NSLAB = _NP // _G
_KSLAB = (FWIDTH - 1 + _G) * NCHAR


def _prep_body(tbl_ref, w80_ref, out_ref):
    tbl = tbl_ref[...]
    p640 = jnp.concatenate(
        [jnp.dot(tbl, w80_ref[pl.ds(CDIM * d, CDIM), :],
                 preferred_element_type=jnp.float32)
         for d in range(FWIDTH)], axis=0)
    cols = []
    for q in range(_G):
        parts = []
        if q:
            parts.append(jnp.zeros((NCHAR * q, FSIZE), jnp.float32))
        parts.append(p640)
        tail = _KSLAB - NCHAR * q - _KDIM
        if tail:
            parts.append(jnp.zeros((tail, FSIZE), jnp.float32))
        cols.append(jnp.concatenate(parts, axis=0))
    out_ref[...] = jnp.concatenate(cols, axis=1)


@jax.jit
def _tc_prep(char_table, w80):
    return pl.pallas_call(
        _prep_body,
        out_shape=jax.ShapeDtypeStruct((_KSLAB, _G * FSIZE), jnp.float32),
    )(char_table, w80)


def _conv_body(ids_ref, p_ref, b_ref, w_ref, out_ref):
    iota = lax.broadcasted_iota(jnp.int32, (1, 1, NCHAR), 2)
    oh = (ids_ref[...][:, :, None] == iota).astype(jnp.bfloat16)
    oh = oh.reshape(_BW, WL * NCHAR)
    m = jnp.concatenate(
        [oh[:, NCHAR * _G * j: NCHAR * _G * j + _KSLAB]
         for j in range(_NSLAB)], axis=0)
    y = jnp.dot(m, p_ref[...], preferred_element_type=jnp.float32)
    y = jnp.max(y.reshape(_NSLAB, _BW, _G * FSIZE), axis=0)
    yq = y[:, :FSIZE]
    for q in range(1, _G):
        yq = jnp.maximum(yq, y[:, FSIZE * q: FSIZE * (q + 1)])
    y = yq
    y = jnp.maximum(y + b_ref[...], 0.0)
    out_ref[...] = jnp.concatenate([w_ref[...], y], axis=1)


def _tc_charconv(cidx, p640, b_row, w_emb, n_words, ids_off, w_off):
    return pl.pallas_call(
        _conv_body,
        out_shape=jax.ShapeDtypeStruct((n_words, EMB + FSIZE), jnp.float32),
        grid=(n_words // _BW,),
        in_specs=[
            pl.BlockSpec((_BW, WL), lambda i: (i + ids_off, 0)),
            pl.BlockSpec((_KSLAB, _G * FSIZE), lambda i: (0, 0)),
            pl.BlockSpec((1, FSIZE), lambda i: (0, 0)),
            pl.BlockSpec((_BW, EMB), lambda i: (i + w_off, 0)),
        ],
        out_specs=pl.BlockSpec((_BW, EMB + FSIZE), lambda i: (i, 0)),
    )(cidx, p640.astype(jnp.bfloat16), b_row, w_emb)


def kernel(doc_w, doc_c, qry_w, qry_c, k_layer, K, W, char_table, conv_w, conv_b):
    widx = jnp.concatenate(
        [doc_w.reshape(-1), qry_w.reshape(-1)]).astype(jnp.int32)
    doc_cidx = doc_c.reshape(-1, WL).astype(jnp.int32)
    qry_cidx = qry_c.reshape(-1, WL).astype(jnp.int32)

    w80 = jnp.transpose(conv_w[:, :, 0, :], (2, 1, 0)).reshape(
        FWIDTH * CDIM, FSIZE)
    b_row = conv_b.reshape(1, FSIZE)

    p640 = _tc_prep(char_table, w80)
    w_emb = _sc_gather(W, widx)
    doc_full = _tc_charconv(doc_cidx, p640, b_row, w_emb, B * DL, 0, 0)
    qry_full = _tc_charconv(qry_cidx, p640, b_row, w_emb, B * QL, 0,
                            B * DL // _BW)

    doc_emb = doc_full.reshape(B, DL, EMB + FSIZE)
    qry_emb = qry_full.reshape(B, QL, EMB + FSIZE)
    return doc_emb, qry_emb
